# baseline (speedup 1.0000x reference)
import functools

import jax
import jax.numpy as jnp
from jax import lax
from jax.experimental import pallas as pl
from jax.experimental.pallas import tpu as pltpu
from jax.experimental.pallas import tpu_sc as plsc

E = 8
TILE = 4096
LANES = 16
NUM_WORKERS = 32


def _logits_body(x_ref, w_ref, lt_ref):
    logits = jnp.dot(x_ref[...], w_ref[...], preferred_element_type=jnp.float32)
    lt_ref[...] = logits.T


def _route_chunk(i, lt_v, wv, iv):
    off = i * LANES
    rows = [lt_v[e, pl.ds(off, LANES)] for e in range(E)]
    m1 = rows[0]
    i1 = jnp.zeros((LANES,), jnp.float32)
    m2 = jnp.full((LANES,), -jnp.inf, jnp.float32)
    i2 = jnp.zeros((LANES,), jnp.float32)
    for e in range(1, E):
        v = rows[e]
        ev = jnp.full((LANES,), float(e), jnp.float32)
        b1 = v > m1
        c2 = v > m2
        m2 = jnp.where(b1, m1, jnp.where(c2, v, m2))
        i2 = jnp.where(b1, i1, jnp.where(c2, ev, i2))
        m1 = jnp.where(b1, v, m1)
        i1 = jnp.where(b1, ev, i1)
    z = rows[0] * jnp.float32(0.0)
    for e in range(E):
        z = z + jnp.exp(rows[e] - m1)
    p1 = 1.0 / z
    p2 = jnp.exp(m2 - m1) / z
    w1 = 1.0 / (1.0 + jnp.exp(p2 - p1))
    wv[0, pl.ds(off, LANES)] = w1
    wv[1, pl.ds(off, LANES)] = 1.0 - w1
    iv[0, pl.ds(off, LANES)] = i1.astype(jnp.int32)
    iv[1, pl.ds(off, LANES)] = i2.astype(jnp.int32)


def _make_router_sc(n_tokens):
    chunk = n_tokens // NUM_WORKERS
    mesh = plsc.VectorSubcoreMesh(core_axis_name="c", subcore_axis_name="s")

    @functools.partial(
        pl.kernel,
        mesh=mesh,
        out_type=[
            jax.ShapeDtypeStruct((2, n_tokens), jnp.float32),
            jax.ShapeDtypeStruct((2, n_tokens), jnp.int32),
        ],
        scratch_types=[
            pltpu.VMEM((E, chunk), jnp.float32),
            pltpu.VMEM((2, chunk), jnp.float32),
            pltpu.VMEM((2, chunk), jnp.int32),
        ],
    )
    def route(lt_hbm, wout_hbm, iout_hbm, lt_v, wv, iv):
        wid = lax.axis_index("s") * 2 + lax.axis_index("c")
        base = wid * chunk
        pltpu.sync_copy(lt_hbm.at[:, pl.ds(base, chunk)], lt_v)

        def body(i, carry):
            _route_chunk(i, lt_v, wv, iv)
            return carry

        lax.fori_loop(0, chunk // LANES, body, 0)
        pltpu.sync_copy(wv, wout_hbm.at[:, pl.ds(base, chunk)])
        pltpu.sync_copy(iv, iout_hbm.at[:, pl.ds(base, chunk)])

    return route


NSEG = 2


def kernel(x, kernel_DE):
    B, T, D = x.shape
    N = B * T
    seg = N // NSEG
    xf = x.reshape(N, D)
    route = _make_router_sc(seg)
    lts = []
    for s in range(NSEG):
        base = s * (seg // TILE)
        lts.append(
            pl.pallas_call(
                _logits_body,
                grid=(seg // TILE,),
                in_specs=[
                    pl.BlockSpec((TILE, D), lambda i, base=base: (base + i, 0)),
                    pl.BlockSpec((D, E), lambda i: (0, 0)),
                ],
                out_specs=pl.BlockSpec((E, TILE), lambda i: (0, i)),
                out_shape=jax.ShapeDtypeStruct((E, seg), jnp.float32),
            )(xf, kernel_DE)
        )
    outs = [route(lt) for lt in lts]
    wout = jnp.concatenate([o[0] for o in outs], axis=1)
    iout = jnp.concatenate([o[1] for o in outs], axis=1)
    return wout.T.reshape(B, T, 2), iout.T.reshape(B, T, 2)


# speedup vs baseline: 1.7675x; 1.7675x over previous
import jax
import jax.numpy as jnp
from jax.experimental import pallas as pl

E = 8
TILE = 4096


def _router_body(x_ref, w_ref, wout_ref, iout_ref):
    logits = jnp.dot(x_ref[...], w_ref[...], preferred_element_type=jnp.float32)
    lt = logits.T

    e_iota = jax.lax.broadcasted_iota(jnp.int32, lt.shape, 0)
    m1 = jnp.max(lt, axis=0, keepdims=True)
    i1 = jnp.min(jnp.where(lt == m1, e_iota, E), axis=0, keepdims=True)
    masked = jnp.where(e_iota == i1, -jnp.inf, lt)
    m2 = jnp.max(masked, axis=0, keepdims=True)
    i2 = jnp.min(jnp.where(masked == m2, e_iota, E), axis=0, keepdims=True)

    z = jnp.sum(jnp.exp(lt - m1), axis=0, keepdims=True)
    p1 = 1.0 / z
    p2 = jnp.exp(m2 - m1) * p1
    w1 = jax.nn.sigmoid(p1 - p2)

    wout_ref[...] = jnp.concatenate([w1, 1.0 - w1], axis=0)
    iout_ref[...] = jnp.concatenate([i1, i2], axis=0)


def kernel(x, kernel_DE):
    B, T, D = x.shape
    N = B * T
    xf = x.reshape(N, D)
    wout, iout = pl.pallas_call(
        _router_body,
        grid=(N // TILE,),
        in_specs=[
            pl.BlockSpec((TILE, D), lambda i: (i, 0)),
            pl.BlockSpec((D, E), lambda i: (0, 0)),
        ],
        out_specs=[
            pl.BlockSpec((2, TILE), lambda i: (0, i)),
            pl.BlockSpec((2, TILE), lambda i: (0, i)),
        ],
        out_shape=[
            jax.ShapeDtypeStruct((2, N), jnp.float32),
            jax.ShapeDtypeStruct((2, N), jnp.int32),
        ],
    )(xf, kernel_DE)
    return wout.T.reshape(B, T, 2), iout.T.reshape(B, T, 2)
